# Initial kernel scaffold; baseline (speedup 1.0000x reference)
import jax
import jax.numpy as jnp
from jax.experimental import pallas as pl


def kernel(predictions):
    raise NotImplementedError("write your pallas kernel here")


# speedup vs baseline: 1.9466x; 1.9466x over previous
import functools

import jax
import jax.numpy as jnp
from jax import lax
from jax.experimental import pallas as pl
from jax.experimental.pallas import tpu as pltpu
from jax.experimental.pallas import tpu_sc as plsc

_B, _T = 16, 4096
_L = 16
_NBLK = _T // _L


def _body(pred_hbm, out_hbm, len_hbm, in_v, out_v, len_v):
    c = lax.axis_index("c")
    s = lax.axis_index("s")
    row = c * 8 + s

    @pl.when(s < 8)
    def _():
        in_v[pl.ds(0, _L)] = jnp.zeros((_L,), jnp.int32)
        pltpu.sync_copy(pred_hbm.at[row], in_v.at[pl.ds(_L, _T)])

        iota1 = lax.iota(jnp.int32, _L) + jnp.ones((_L,), jnp.int32)
        zero_v = jnp.zeros((_L,), jnp.int32)
        one_v = jnp.ones((_L,), jnp.int32)

        def step(i, cnt_vec):
            x = in_v[pl.ds(_L + i * _L, _L)]
            xp = in_v[pl.ds(_L - 1 + i * _L, _L)]
            m = (x != xp) & (x != zero_v)
            mi = jnp.where(m, one_v, zero_v)
            cum = plsc.cumsum(mi)
            pc = plsc.all_reduce_population_count(m)
            kept_dest = cnt_vec + cum - one_v
            top = jnp.full((_L,), _T - i * _L, jnp.int32)
            drop_dest = top + cnt_vec - iota1 + cum
            dest = jnp.where(m, kept_dest, drop_dest)
            val = jnp.where(m, x, zero_v)
            plsc.store_scatter(out_v, [dest], val)
            return cnt_vec + pc

        cnt_vec = lax.fori_loop(0, _NBLK, step, jnp.zeros((_L,), jnp.int32))

        len_v[...] = cnt_vec
        pltpu.sync_copy(out_v, out_hbm.at[row])
        pltpu.sync_copy(len_v, len_hbm.at[row])


@jax.jit
def _run(predictions):
    mesh = plsc.VectorSubcoreMesh(
        core_axis_name="c", subcore_axis_name="s", num_cores=2, num_subcores=16
    )
    k = pl.kernel(
        _body,
        out_type=[
            jax.ShapeDtypeStruct((_B, _T), jnp.int32),
            jax.ShapeDtypeStruct((_B, _L), jnp.int32),
        ],
        mesh=mesh,
        scratch_types=[
            pltpu.VMEM((_L + _T,), jnp.int32),
            pltpu.VMEM((_T,), jnp.int32),
            pltpu.VMEM((_L,), jnp.int32),
        ],
        compiler_params=pltpu.CompilerParams(
            needs_layout_passes=False, use_tc_tiling_on_sc=False
        ),
    )
    compact, lens = k(predictions)
    return compact, lens[:, 0]


def kernel(predictions):
    return _run(predictions)
